# baseline (speedup 1.0000x reference)
import functools

import jax
import jax.numpy as jnp
from jax import lax
from jax.experimental import pallas as pl
from jax.experimental.pallas import tpu as pltpu
from jax.experimental.pallas import tpu_sc as plsc

B = 16384
F = 32
V = 1000000
BLK = 1024
NBLK = (V + BLK - 1) // BLK
LASTB = NBLK - 1
LAST_FULL = (V // 128) * 128 - LASTB * BLK
LAST_TAIL = V - LASTB * BLK - LAST_FULL
SCR = B + 32


def _make_kernels():
    info = plsc.get_sparse_core_info()
    NC, NS = info.num_cores, info.num_subcores
    NW = NC * NS
    BPW = B // NW
    TPW = (NBLK + NW - 1) // NW

    mesh = plsc.VectorSubcoreMesh(core_axis_name="c", subcore_axis_name="s")

    @functools.partial(
        pl.kernel,
        mesh=mesh,
        compiler_params=pltpu.CompilerParams(needs_layout_passes=False),
        out_type=(
            jax.ShapeDtypeStruct((SCR, 128), jnp.float32),
            jax.ShapeDtypeStruct((SCR, 128), jnp.float32),
        ),
        scratch_types=[
            pltpu.VMEM((B,), jnp.int32),
            pltpu.VMEM((B + 16,), jnp.int32),
            pltpu.VMEM((B + 16,), jnp.int32),
            pltpu.VMEM((2, F, BLK), jnp.float32),
            pltpu.VMEM((272,), jnp.int32),
            pltpu.VMEM((272,), jnp.int32),
            pltpu.VMEM((4, 16, 128), jnp.float32),
            pltpu.SMEM((1,), jnp.int32),
            pltpu.SemaphoreType.DMA,
            pltpu.SemaphoreType.DMA,
            pltpu.SemaphoreType.DMA,
        ],
    )
    def gather_k(user_hbm, item_hbm, utt_hbm, itt_hbm, us_hbm, is_hbm,
                 idx_v, ids_v, rs_v, win, bl_i, bl_r, stg, kref,
                 semw0, semw1, sems):
        wid = lax.axis_index("s") * NC + lax.axis_index("c")
        lanes = lax.iota(jnp.int32, 16)
        c_lo = lanes
        c_hi = lanes + 16

        def run_pass(idx_hbm, tab_hbm, scr_hbm):
            def fetchw(t, par_is_odd):
                sem = semw1 if par_is_odd else semw0
                par = 1 if par_is_odd else 0
                b = wid + NW * t
                r0 = pl.multiple_of(b * BLK, 128)

                @pl.when(b < LASTB)
                def _():
                    pltpu.async_copy(
                        tab_hbm.at[:, pl.ds(r0, BLK)], win.at[par], sem
                    )

                @pl.when(b == LASTB)
                def _():
                    pltpu.async_copy(
                        tab_hbm.at[:, pl.ds(LASTB * BLK, LAST_FULL)],
                        win.at[par, :, pl.ds(0, LAST_FULL)],
                        sem,
                    )
                    off_tail = pl.multiple_of(
                        (b - LASTB) + LASTB * BLK + LAST_FULL, 128
                    )
                    pltpu.async_copy(
                        tab_hbm.at[:, pl.ds(off_tail, 128)],
                        win.at[par, :, pl.ds(LAST_FULL, 128)],
                        sem,
                    )

            def drainw(t, par_is_odd):
                sem = semw1 if par_is_odd else semw0
                par = 1 if par_is_odd else 0
                b = wid + NW * t

                @pl.when(b < LASTB)
                def _():
                    pltpu.make_async_copy(
                        tab_hbm.at[:, pl.ds(0, BLK)], win.at[par], sem
                    ).wait()

                @pl.when(b == LASTB)
                def _():
                    pltpu.make_async_copy(
                        tab_hbm.at[:, pl.ds(0, LAST_FULL)],
                        win.at[par, :, pl.ds(0, LAST_FULL)],
                        sem,
                    ).wait()
                    pltpu.make_async_copy(
                        tab_hbm.at[:, pl.ds(0, 128)],
                        win.at[par, :, pl.ds(LAST_FULL, 128)],
                        sem,
                    ).wait()

            dump = B + lanes
            for s in range(4):
                pltpu.async_copy(stg.at[s], scr_hbm.at[dump], sems)
            kref[0] = 0

            def drain_one_scatter():
                pltpu.make_async_copy(
                    stg.at[0], scr_hbm.at[pl.ds(B, 16)], sems
                ).wait()

            fetchw(jnp.int32(0), False)
            fetchw(jnp.int32(1), True)

            pltpu.sync_copy(idx_hbm, idx_v)

            def scan(ch, n):
                r = idx_v[pl.ds(ch * 16, 16)]
                m = ((r >> 10) & (NW - 1)) == wid
                plsc.store_compressed(
                    ids_v.at[pl.ds(n, 16)], ch * 16 + lanes, mask=m
                )
                plsc.store_compressed(rs_v.at[pl.ds(n, 16)], r, mask=m)
                return n + plsc.all_reduce_population_count(m)[0]

            n = lax.fori_loop(0, B // 16, scan, jnp.int32(0))
            nseg = (n + 255) >> 8

            def visit(t, carry):
                b = wid + NW * t
                par_odd = (t & 1) == 1

                @pl.when((t & 1) == 0)
                def _():
                    drainw(t, False)

                @pl.when((t & 1) == 1)
                def _():
                    drainw(t, True)

                parv = jnp.full((16,), t & 1, jnp.int32)

                def seg(s, carry2):
                    def seg_scan(c2, nb):
                        off = s * 256 + c2 * 16
                        idv = ids_v[pl.ds(off, 16)]
                        rv = rs_v[pl.ds(off, 16)]
                        m = ((rv >> 10) == b) & ((off + lanes) < n)
                        plsc.store_compressed(
                            bl_i.at[pl.ds(nb, 16)], idv, mask=m
                        )
                        plsc.store_compressed(
                            bl_r.at[pl.ds(nb, 16)], rv, mask=m
                        )
                        return nb + plsc.all_reduce_population_count(m)[0]

                    nb = lax.fori_loop(0, 16, seg_scan, jnp.int32(0))
                    nchb = (nb + 15) >> 4

                    def ext(c3, carry3):
                        idv = bl_i[pl.ds(c3 * 16, 16)]
                        rv = bl_r[pl.ds(c3 * 16, 16)]
                        valid = (c3 * 16 + lanes) < nb
                        safe = jnp.where(valid, idv, dump)
                        k = kref[0]
                        slot = k & 3
                        drain_one_scatter()
                        for e in range(16):
                            rr = jnp.full((16,), rv[e] & (BLK - 1),
                                          jnp.int32)
                            g_lo = plsc.load_gather(win, [parv, c_lo, rr])
                            g_hi = plsc.load_gather(win, [parv, c_hi, rr])
                            stg[slot, e, pl.ds(0, 16)] = g_lo
                            stg[slot, e, pl.ds(16, 16)] = g_hi
                        pltpu.async_copy(
                            stg.at[slot], scr_hbm.at[safe], sems
                        )
                        kref[0] = k + 1
                        return carry3

                    lax.fori_loop(0, nchb, ext, 0)
                    return carry2

                lax.fori_loop(0, nseg, seg, 0)

                @pl.when((t & 1) == 0)
                def _():
                    fetchw(t + 2, False)

                @pl.when((t & 1) == 1)
                def _():
                    fetchw(t + 2, True)

                return carry

            def visit_guard(t, carry):
                @pl.when(wid + NW * t < NBLK)
                def _():
                    visit(t, 0)

                return carry

            lax.fori_loop(0, TPW, visit_guard, 0)
            for s in range(4):
                drain_one_scatter()

        run_pass(user_hbm, utt_hbm, us_hbm)
        run_pass(item_hbm, itt_hbm, is_hbm)

    @functools.partial(
        pl.kernel,
        mesh=mesh,
        compiler_params=pltpu.CompilerParams(needs_layout_passes=False),
        out_type=jax.ShapeDtypeStruct((B,), jnp.float32),
        scratch_types=[
            pltpu.VMEM((256, 128), jnp.float32),
            pltpu.VMEM((256, 128), jnp.float32),
            pltpu.VMEM((F,), jnp.float32),
            pltpu.VMEM((16,), jnp.float32),
            pltpu.VMEM((BPW,), jnp.float32),
        ],
    )
    def dot_k(us_hbm, is_hbm, w_hbm, b_hbm, out_hbm,
              ubuf, ibuf, w_v, b_v, out_v):
        wid = lax.axis_index("s") * NC + lax.axis_index("c")
        base = wid * BPW
        pltpu.sync_copy(w_hbm, w_v)
        pltpu.sync_copy(b_hbm, b_v)
        w_lo = w_v[pl.ds(0, 16)]
        w_hi = w_v[pl.ds(16, 16)]
        ws = [w_lo[c] for c in range(16)] + [w_hi[c] for c in range(16)]
        bvec = b_v[...]
        lane = lax.iota(jnp.int32, 16)

        for h in range(2):
            pltpu.sync_copy(
                us_hbm.at[pl.ds(base + h * 256, 256)], ubuf
            )
            pltpu.sync_copy(
                is_hbm.at[pl.ds(base + h * 256, 256)], ibuf
            )

            def group(g, carry):
                rows = g * 16 + lane
                acc = bvec
                for c in range(F):
                    cv = jnp.full((16,), c, jnp.int32)
                    gu = plsc.load_gather(ubuf, [rows, cv])
                    gv = plsc.load_gather(ibuf, [rows, cv])
                    acc = acc + gu * gv * ws[c]
                out_v[pl.ds(h * 256 + g * 16, 16)] = 1.0 / (
                    1.0 + jnp.exp(-acc)
                )
                return carry

            lax.fori_loop(0, 16, group, 0)

        pltpu.sync_copy(out_v, out_hbm.at[pl.ds(base, BPW)])

    return gather_k, dot_k


_gather_k, _dot_k = _make_kernels()


def kernel(user, item, user_table, item_table, W, b):
    us, is_ = _gather_k(user, item, user_table.T, item_table.T)
    return _dot_k(us, is_, W.reshape(F), jnp.broadcast_to(b, (16,)))


# speedup vs baseline: 2.3013x; 2.3013x over previous
import functools

import jax
import jax.numpy as jnp
from jax import lax
from jax.experimental import pallas as pl
from jax.experimental.pallas import tpu as pltpu
from jax.experimental.pallas import tpu_sc as plsc

B = 16384
F = 32
V = 1000000
BLK = 1024
NBLK = (V + BLK - 1) // BLK
LASTB = NBLK - 1
LAST_FULL = (V // 128) * 128 - LASTB * BLK
LAST_TAIL = V - LASTB * BLK - LAST_FULL
SCR = B + 512


def _make_kernels():
    info = plsc.get_sparse_core_info()
    NC, NS = info.num_cores, info.num_subcores
    NW = NC * NS
    BPW = B // NW
    TPW = (NBLK + NW - 1) // NW

    mesh = plsc.VectorSubcoreMesh(core_axis_name="c", subcore_axis_name="s")

    @functools.partial(
        pl.kernel,
        mesh=mesh,
        compiler_params=pltpu.CompilerParams(needs_layout_passes=False),
        out_type=(
            jax.ShapeDtypeStruct((SCR, 128), jnp.float32),
            jax.ShapeDtypeStruct((SCR, 128), jnp.float32),
        ),
        scratch_types=[
            pltpu.VMEM((B,), jnp.int32),
            pltpu.VMEM((B + 16,), jnp.int32),
            pltpu.VMEM((B + 16,), jnp.int32),
            pltpu.VMEM((2, F, BLK), jnp.float32),
            pltpu.VMEM((272,), jnp.int32),
            pltpu.VMEM((272,), jnp.int32),
            pltpu.VMEM((4, 16, 128), jnp.float32),
            pltpu.SMEM((1,), jnp.int32),
            pltpu.SemaphoreType.DMA,
            pltpu.SemaphoreType.DMA,
            pltpu.SemaphoreType.DMA,
        ],
    )
    def gather_k(user_hbm, item_hbm, utt_hbm, itt_hbm, us_hbm, is_hbm,
                 idx_v, ids_v, rs_v, win, bl_i, bl_r, stg, kref,
                 semw0, semw1, sems):
        wid = lax.axis_index("s") * NC + lax.axis_index("c")
        lanes = lax.iota(jnp.int32, 16)
        c_lo = lanes
        c_hi = lanes + 16

        def run_pass(idx_hbm, tab_hbm, scr_hbm):
            def fetchw(t, par_is_odd):
                sem = semw1 if par_is_odd else semw0
                par = 1 if par_is_odd else 0
                b = wid + NW * t
                r0 = pl.multiple_of(b * BLK, 128)

                @pl.when(b < LASTB)
                def _():
                    pltpu.async_copy(
                        tab_hbm.at[:, pl.ds(r0, BLK)], win.at[par], sem
                    )

                @pl.when(b == LASTB)
                def _():
                    pltpu.async_copy(
                        tab_hbm.at[:, pl.ds(LASTB * BLK, LAST_FULL)],
                        win.at[par, :, pl.ds(0, LAST_FULL)],
                        sem,
                    )
                    off_tail = pl.multiple_of(
                        (b - LASTB) + LASTB * BLK + LAST_FULL, 128
                    )
                    pltpu.async_copy(
                        tab_hbm.at[:, pl.ds(off_tail, 128)],
                        win.at[par, :, pl.ds(LAST_FULL, 128)],
                        sem,
                    )

            def drainw(t, par_is_odd):
                sem = semw1 if par_is_odd else semw0
                par = 1 if par_is_odd else 0
                b = wid + NW * t

                @pl.when(b < LASTB)
                def _():
                    pltpu.make_async_copy(
                        tab_hbm.at[:, pl.ds(0, BLK)], win.at[par], sem
                    ).wait()

                @pl.when(b == LASTB)
                def _():
                    pltpu.make_async_copy(
                        tab_hbm.at[:, pl.ds(0, LAST_FULL)],
                        win.at[par, :, pl.ds(0, LAST_FULL)],
                        sem,
                    ).wait()
                    pltpu.make_async_copy(
                        tab_hbm.at[:, pl.ds(0, 128)],
                        win.at[par, :, pl.ds(LAST_FULL, 128)],
                        sem,
                    ).wait()

            dump = B + wid * 16 + lanes
            for s in range(4):
                pltpu.async_copy(stg.at[s], scr_hbm.at[dump], sems)
            kref[0] = 0

            def drain_one_scatter():
                pltpu.make_async_copy(
                    stg.at[0], scr_hbm.at[pl.ds(B, 16)], sems
                ).wait()

            fetchw(jnp.int32(0), False)
            fetchw(jnp.int32(1), True)

            pltpu.sync_copy(idx_hbm, idx_v)

            def scan(ch, n):
                r = idx_v[pl.ds(ch * 16, 16)]
                m = ((r >> 10) & (NW - 1)) == wid
                plsc.store_compressed(
                    ids_v.at[pl.ds(n, 16)], ch * 16 + lanes, mask=m
                )
                plsc.store_compressed(rs_v.at[pl.ds(n, 16)], r, mask=m)
                return n + plsc.all_reduce_population_count(m)[0]

            n = lax.fori_loop(0, B // 16, scan, jnp.int32(0))
            nseg = (n + 255) >> 8

            def visit(t, carry):
                b = wid + NW * t
                par_odd = (t & 1) == 1

                @pl.when((t & 1) == 0)
                def _():
                    drainw(t, False)

                @pl.when((t & 1) == 1)
                def _():
                    drainw(t, True)

                parv = jnp.full((16,), t & 1, jnp.int32)

                def seg(s, carry2):
                    def seg_scan(c2, nb):
                        off = s * 256 + c2 * 16
                        idv = ids_v[pl.ds(off, 16)]
                        rv = rs_v[pl.ds(off, 16)]
                        m = ((rv >> 10) == b) & ((off + lanes) < n)
                        plsc.store_compressed(
                            bl_i.at[pl.ds(nb, 16)], idv, mask=m
                        )
                        plsc.store_compressed(
                            bl_r.at[pl.ds(nb, 16)], rv, mask=m
                        )
                        return nb + plsc.all_reduce_population_count(m)[0]

                    nb = lax.fori_loop(0, 16, seg_scan, jnp.int32(0))
                    nchb = (nb + 15) >> 4

                    def ext(c3, carry3):
                        idv = bl_i[pl.ds(c3 * 16, 16)]
                        rv = bl_r[pl.ds(c3 * 16, 16)]
                        valid = (c3 * 16 + lanes) < nb
                        safe = jnp.where(valid, idv, dump)
                        k = kref[0]
                        slot = k & 3
                        drain_one_scatter()
                        for e in range(16):
                            rr = jnp.full((16,), rv[e] & (BLK - 1),
                                          jnp.int32)
                            g_lo = plsc.load_gather(win, [parv, c_lo, rr])
                            g_hi = plsc.load_gather(win, [parv, c_hi, rr])
                            stg[slot, e, pl.ds(0, 16)] = g_lo
                            stg[slot, e, pl.ds(16, 16)] = g_hi
                        pltpu.async_copy(
                            stg.at[slot], scr_hbm.at[safe], sems
                        )
                        kref[0] = k + 1
                        return carry3

                    lax.fori_loop(0, nchb, ext, 0)
                    return carry2

                lax.fori_loop(0, nseg, seg, 0)

                @pl.when((t & 1) == 0)
                def _():
                    fetchw(t + 2, False)

                @pl.when((t & 1) == 1)
                def _():
                    fetchw(t + 2, True)

                return carry

            def visit_guard(t, carry):
                @pl.when(wid + NW * t < NBLK)
                def _():
                    visit(t, 0)

                return carry

            lax.fori_loop(0, TPW, visit_guard, 0)
            for s in range(4):
                drain_one_scatter()

        run_pass(user_hbm, utt_hbm, us_hbm)
        run_pass(item_hbm, itt_hbm, is_hbm)

    @functools.partial(
        pl.kernel,
        mesh=mesh,
        compiler_params=pltpu.CompilerParams(needs_layout_passes=False),
        out_type=jax.ShapeDtypeStruct((B,), jnp.float32),
        scratch_types=[
            pltpu.VMEM((256, 128), jnp.float32),
            pltpu.VMEM((256, 128), jnp.float32),
            pltpu.VMEM((F,), jnp.float32),
            pltpu.VMEM((16,), jnp.float32),
            pltpu.VMEM((BPW,), jnp.float32),
        ],
    )
    def dot_k(us_hbm, is_hbm, w_hbm, b_hbm, out_hbm,
              ubuf, ibuf, w_v, b_v, out_v):
        wid = lax.axis_index("s") * NC + lax.axis_index("c")
        base = wid * BPW
        pltpu.sync_copy(w_hbm, w_v)
        pltpu.sync_copy(b_hbm, b_v)
        w_lo = w_v[pl.ds(0, 16)]
        w_hi = w_v[pl.ds(16, 16)]
        ws = [w_lo[c] for c in range(16)] + [w_hi[c] for c in range(16)]
        bvec = b_v[...]
        lane = lax.iota(jnp.int32, 16)

        for h in range(2):
            pltpu.sync_copy(
                us_hbm.at[pl.ds(base + h * 256, 256)], ubuf
            )
            pltpu.sync_copy(
                is_hbm.at[pl.ds(base + h * 256, 256)], ibuf
            )

            def group(g, carry):
                rows = g * 16 + lane
                acc = bvec
                for c in range(F):
                    cv = jnp.full((16,), c, jnp.int32)
                    gu = plsc.load_gather(ubuf, [rows, cv])
                    gv = plsc.load_gather(ibuf, [rows, cv])
                    acc = acc + gu * gv * ws[c]
                out_v[pl.ds(h * 256 + g * 16, 16)] = 1.0 / (
                    1.0 + jnp.exp(-acc)
                )
                return carry

            lax.fori_loop(0, 16, group, 0)

        pltpu.sync_copy(out_v, out_hbm.at[pl.ds(base, BPW)])

    return gather_k, dot_k


_gather_k, _dot_k = _make_kernels()


def kernel(user, item, user_table, item_table, W, b):
    us, is_ = _gather_k(user, item, user_table.T, item_table.T)
    return _dot_k(us, is_, W.reshape(F), jnp.broadcast_to(b, (16,)))
